# baseline (speedup 1.0000x reference)
import functools

import jax
import jax.numpy as jnp
from jax import lax
from jax.experimental import pallas as pl
from jax.experimental.pallas import tpu as pltpu
from jax.experimental.pallas import tpu_sc as plsc

N = 10000
NP = 10240
E = 320000
NFEAT = 128
NHID = 128
DIM_MP = 64
EDGE_DIM = 16
NMETA = 2
NCLASS = 8

_NC = 2
_NS = 16
_EP = 327680
_EPW = _EP // _NS
_CH = 64
_BLKE = 2048
_NCHB = _BLKE // _CH
_NBLK = _EPW // _BLKE
_NRS = NP // _NS


def _prep_body(x_ref, w_ref, ad_ref, as_ref,
               h_ref, hd_ref, hs_ref, mhd_ref, mhs_ref):
    i = pl.program_id(1)
    h = jnp.dot(x_ref[...], w_ref[0], preferred_element_type=jnp.float32)
    h_ref[0] = h
    hd = jnp.dot(h, ad_ref[0], preferred_element_type=jnp.float32)
    hs = jnp.dot(h, as_ref[0], preferred_element_type=jnp.float32)
    hd_ref[0] = hd
    hs_ref[0] = hs
    bmhd = jnp.max(hd).reshape(1, 1, 1)
    bmhs = jnp.max(hs).reshape(1, 1, 1)

    @pl.when(i == 0)
    def _():
        mhd_ref[...] = bmhd
        mhs_ref[...] = bmhs

    @pl.when(i > 0)
    def _():
        mhd_ref[...] = jnp.maximum(mhd_ref[...], bmhd)
        mhs_ref[...] = jnp.maximum(mhs_ref[...], bmhs)


def _prep_tc(x, w, a_d, a_s):
    f = x.shape[1]
    nh = w.shape[2]
    br = 2048
    grid = (2, NP // br)
    return pl.pallas_call(
        _prep_body,
        grid=grid,
        in_specs=[
            pl.BlockSpec((br, f), lambda m, i: (i, 0)),
            pl.BlockSpec((1, f, nh), lambda m, i: (m, 0, 0)),
            pl.BlockSpec((1, nh, 1), lambda m, i: (m, 0, 0)),
            pl.BlockSpec((1, nh, 1), lambda m, i: (m, 0, 0)),
        ],
        out_specs=[
            pl.BlockSpec((1, br, nh), lambda m, i: (m, i, 0)),
            pl.BlockSpec((1, br, 1), lambda m, i: (m, i, 0)),
            pl.BlockSpec((1, br, 1), lambda m, i: (m, i, 0)),
            pl.BlockSpec((1, 1, 1), lambda m, i: (m, 0, 0)),
            pl.BlockSpec((1, 1, 1), lambda m, i: (m, 0, 0)),
        ],
        out_shape=[
            jax.ShapeDtypeStruct((2, NP, nh), jnp.float32),
            jax.ShapeDtypeStruct((2, NP, 1), jnp.float32),
            jax.ShapeDtypeStruct((2, NP, 1), jnp.float32),
            jax.ShapeDtypeStruct((2, 1, 1), jnp.float32),
            jax.ShapeDtypeStruct((2, 1, 1), jnp.float32),
        ],
    )(x, w, a_d, a_s)


def _next_body(n_ref, d_ref, w_ref, ad_ref, as_ref,
               h_ref, hd_ref, hs_ref, mhd_ref, mhs_ref):
    i = pl.program_id(1)
    x = n_ref[0] / (d_ref[0] + 1e-16)
    x = jnp.where(x > 0, x, jnp.exp(x) - 1.0)
    h = jnp.dot(x, w_ref[0], preferred_element_type=jnp.float32)
    h_ref[0] = h
    hd = jnp.dot(h, ad_ref[0], preferred_element_type=jnp.float32)
    hs = jnp.dot(h, as_ref[0], preferred_element_type=jnp.float32)
    hd_ref[0] = hd
    hs_ref[0] = hs
    bmhd = jnp.max(hd).reshape(1, 1, 1)
    bmhs = jnp.max(hs).reshape(1, 1, 1)

    @pl.when(i == 0)
    def _():
        mhd_ref[...] = bmhd
        mhs_ref[...] = bmhs

    @pl.when(i > 0)
    def _():
        mhd_ref[...] = jnp.maximum(mhd_ref[...], bmhd)
        mhs_ref[...] = jnp.maximum(mhs_ref[...], bmhs)


def _next_tc(num, den, w, a_d, a_s):
    hin = num.shape[2]
    nh = w.shape[2]
    br = 2048
    grid = (2, NP // br)
    return pl.pallas_call(
        _next_body,
        grid=grid,
        in_specs=[
            pl.BlockSpec((1, br, hin), lambda m, i: (m, i, 0)),
            pl.BlockSpec((1, br, 1), lambda m, i: (m, i, 0)),
            pl.BlockSpec((1, hin, nh), lambda m, i: (m, 0, 0)),
            pl.BlockSpec((1, nh, 1), lambda m, i: (m, 0, 0)),
            pl.BlockSpec((1, nh, 1), lambda m, i: (m, 0, 0)),
        ],
        out_specs=[
            pl.BlockSpec((1, br, nh), lambda m, i: (m, i, 0)),
            pl.BlockSpec((1, br, 1), lambda m, i: (m, i, 0)),
            pl.BlockSpec((1, br, 1), lambda m, i: (m, i, 0)),
            pl.BlockSpec((1, 1, 1), lambda m, i: (m, 0, 0)),
            pl.BlockSpec((1, 1, 1), lambda m, i: (m, 0, 0)),
        ],
        out_shape=[
            jax.ShapeDtypeStruct((2, NP, nh), jnp.float32),
            jax.ShapeDtypeStruct((2, NP, 1), jnp.float32),
            jax.ShapeDtypeStruct((2, NP, 1), jnp.float32),
            jax.ShapeDtypeStruct((2, 1, 1), jnp.float32),
            jax.ShapeDtypeStruct((2, 1, 1), jnp.float32),
        ],
    )(num, den, w, a_d, a_s)


def _edge_body(ee_ref, a1_ref, a2_ref, d1_ref, d2_ref, m1_ref, m2_ref):
    i = pl.program_id(0)
    d1 = jnp.dot(ee_ref[...], a1_ref[...], preferred_element_type=jnp.float32)
    d2 = jnp.dot(ee_ref[...], a2_ref[...], preferred_element_type=jnp.float32)
    d1_ref[...] = d1
    d2_ref[...] = d2
    b1 = jnp.max(d1).reshape(1, 1)
    b2 = jnp.max(d2).reshape(1, 1)

    @pl.when(i == 0)
    def _():
        m1_ref[...] = b1
        m2_ref[...] = b2

    @pl.when(i > 0)
    def _():
        m1_ref[...] = jnp.maximum(m1_ref[...], b1)
        m2_ref[...] = jnp.maximum(m2_ref[...], b2)


def _edge_tc(ee, a_e1, a_e2):
    be = 2000
    grid = E // be
    return pl.pallas_call(
        _edge_body,
        grid=(grid,),
        in_specs=[
            pl.BlockSpec((be, EDGE_DIM), lambda i: (i, 0)),
            pl.BlockSpec((EDGE_DIM, 1), lambda i: (0, 0)),
            pl.BlockSpec((EDGE_DIM, 1), lambda i: (0, 0)),
        ],
        out_specs=[
            pl.BlockSpec((be, 1), lambda i: (i, 0)),
            pl.BlockSpec((be, 1), lambda i: (i, 0)),
            pl.BlockSpec((1, 1), lambda i: (0, 0)),
            pl.BlockSpec((1, 1), lambda i: (0, 0)),
        ],
        out_shape=[
            jax.ShapeDtypeStruct((E, 1), jnp.float32),
            jax.ShapeDtypeStruct((E, 1), jnp.float32),
            jax.ShapeDtypeStruct((1, 1), jnp.float32),
            jax.ShapeDtypeStruct((1, 1), jnp.float32),
        ],
    )(ee, a_e1, a_e2)


def _sc_body(hh,
             h_ref, hd_ref, hs_ref, src_ref, dst_ref, eed_ref,
             cv_ref, num_ref, den_ref,
             hd_v, hs_v, cv, srcb, dstb, eedb, dstc, wc, rows0, rows1,
             zden, num_sp, den_sp, gsem0, gsem1, ssem, dsem):
    ci = lax.axis_index("c")
    s = lax.axis_index("s")

    pltpu.sync_copy(hd_ref.at[ci], hd_v)
    pltpu.sync_copy(hs_ref.at[ci], hs_v)
    pltpu.sync_copy(cv_ref.at[ci], cv)

    nfv = hh // 16
    ngr = _CH // 16

    @plsc.parallel_loop(0, _CH)
    def _(r):
        for f in range(nfv):
            rows0[r, pl.ds(f * 16, 16)] = jnp.zeros((16,), jnp.float32)

    @plsc.parallel_loop(0, _NRS, step=16)
    def _(r):
        zden[pl.ds(r, 16)] = jnp.zeros((16,), jnp.float32)

    for k in range(_NRS // _CH):
        pltpu.sync_copy(rows0, num_sp.at[pl.ds(s * _NRS + k * _CH, _CH)])
    pltpu.sync_copy(zden, den_sp.at[pl.ds(s * _NRS, _NRS)])

    plsc.subcore_barrier()

    cvv = cv[...]
    wbase = s * _EPW
    coff = ci * NP

    def score(c):
        off = c * _CH
        for g in range(ngr):
            o = off + g * 16
            dv = dstb[pl.ds(o, 16)]
            sv = srcb[pl.ds(o, 16)] - coff
            ev = eedb[pl.ds(o, 16)]
            hdv = plsc.load_gather(hd_v, [dv])
            hsv = plsc.load_gather(hs_v, [sv])
            t = hdv + hsv + ev
            e = jnp.maximum(t, t * 0.2)
            w = jnp.exp(e - cvv)
            wc[c, pl.ds(g * 16, 16)] = w
            dstc[c, pl.ds(g * 16, 16)] = dv

    def gissue(c, rbuf, gsem):
        pltpu.async_copy(h_ref.at[srcb.at[pl.ds(c * _CH, _CH)]], rbuf, gsem)

    def scale(c, rbuf):
        @plsc.parallel_loop(0, ngr)
        def _(g):
            wv16 = wc[c, pl.ds(g * 16, 16)]
            for j in range(16):
                wb = jnp.full((16,), wv16[j], jnp.float32)
                r = g * 16 + j
                for f in range(nfv):
                    sl = pl.ds(f * 16, 16)
                    rbuf[r, sl] = rbuf[r, sl] * wb

    def swait(rbuf):
        pltpu.make_async_copy(rbuf, num_sp.at[dstc.at[0]], ssem).wait()

    def dwait():
        pltpu.make_async_copy(wc.at[0], den_sp.at[dstc.at[0]], dsem).wait()

    def block_body(b, carry):
        @pl.when(b > 0)
        def _():
            swait(rows0)
            dwait()

        base = wbase + b * _BLKE
        pltpu.sync_copy(src_ref.at[ci, pl.ds(base, _BLKE)], srcb)
        pltpu.sync_copy(dst_ref.at[ci, pl.ds(base, _BLKE)], dstb)
        pltpu.sync_copy(eed_ref.at[ci, pl.ds(base, _BLKE)], eedb)

        score(0)
        gissue(0, rows0, gsem0)

        def chunk_body(c, carry):
            even = (c & 1) == 0

            @pl.when(c >= 1)
            def _():
                swait(rows0)
                dwait()

            @pl.when(c < _NCHB - 1)
            def _():
                score(c + 1)

                @pl.when(even)
                def _():
                    gissue(c + 1, rows1, gsem1)

                @pl.when(jnp.logical_not(even))
                def _():
                    gissue(c + 1, rows0, gsem0)

            @pl.when(even)
            def _():
                pltpu.make_async_copy(
                    h_ref.at[srcb.at[pl.ds(0, _CH)]], rows0, gsem0).wait()
                scale(c, rows0)
                pltpu.async_copy(rows0, num_sp.at[dstc.at[c]], ssem, add=True)

            @pl.when(jnp.logical_not(even))
            def _():
                pltpu.make_async_copy(
                    h_ref.at[srcb.at[pl.ds(0, _CH)]], rows1, gsem1).wait()
                scale(c, rows1)
                pltpu.async_copy(rows1, num_sp.at[dstc.at[c]], ssem, add=True)

            pltpu.async_copy(wc.at[c], den_sp.at[dstc.at[c]], dsem, add=True)
            return carry

        lax.fori_loop(0, _NCHB, chunk_body, 0)
        return carry

    lax.fori_loop(0, _NBLK, block_body, 0)

    swait(rows0)
    dwait()

    plsc.subcore_barrier()

    r0 = s * _NRS
    pltpu.sync_copy(num_sp.at[pl.ds(r0, _NRS)], num_ref.at[ci, pl.ds(r0, _NRS)])
    pltpu.sync_copy(den_sp.at[pl.ds(r0, _NRS)], den_ref.at[ci, pl.ds(r0, _NRS)])


def _sc_aggregate(h, hd, hs, src, dst, eed, cvec):
    hh = h.shape[2]
    mesh = plsc.VectorSubcoreMesh(core_axis_name="c", subcore_axis_name="s")
    f = pl.kernel(
        functools.partial(_sc_body, hh),
        out_type=[
            jax.ShapeDtypeStruct((2, NP, hh), jnp.float32),
            jax.ShapeDtypeStruct((2, NP), jnp.float32),
        ],
        mesh=mesh,
        scratch_types=[
            pltpu.VMEM((NP,), jnp.float32),
            pltpu.VMEM((NP,), jnp.float32),
            pltpu.VMEM((16,), jnp.float32),
            pltpu.VMEM((_BLKE,), jnp.int32),
            pltpu.VMEM((_BLKE,), jnp.int32),
            pltpu.VMEM((_BLKE,), jnp.float32),
            pltpu.VMEM((_NCHB, _CH), jnp.int32),
            pltpu.VMEM((_NCHB, _CH), jnp.float32),
            pltpu.VMEM((_CH, hh), jnp.float32),
            pltpu.VMEM((_CH, hh), jnp.float32),
            pltpu.VMEM((_NRS,), jnp.float32),
            pltpu.VMEM_SHARED((NP, hh), jnp.float32),
            pltpu.VMEM_SHARED((NP,), jnp.float32),
            pltpu.SemaphoreType.DMA,
            pltpu.SemaphoreType.DMA,
            pltpu.SemaphoreType.DMA,
            pltpu.SemaphoreType.DMA,
        ],
        compiler_params=pltpu.CompilerParams(
            needs_layout_passes=False, use_tc_tiling_on_sc=False),
    )
    return f(h.reshape(2 * NP, hh), hd, hs, src, dst, eed, cvec)


def _final_body(n_ref, d_ref, aw_ref, lw_ref, lb_ref, out_ref):
    x0 = n_ref[0] / (d_ref[0] + 1e-16)
    x0 = jnp.where(x0 > 0, x0, jnp.exp(x0) - 1.0)
    x1 = n_ref[1] / (d_ref[1] + 1e-16)
    x1 = jnp.where(x1 > 0, x1, jnp.exp(x1) - 1.0)
    s0 = jnp.tanh(jnp.dot(x0, aw_ref[...], preferred_element_type=jnp.float32))
    s1 = jnp.tanh(jnp.dot(x1, aw_ref[...], preferred_element_type=jnp.float32))
    m = jnp.maximum(s0, s1)
    b0 = jnp.exp(s0 - m)
    b1 = jnp.exp(s1 - m)
    tot = b0 + b1
    outp = (b0 / tot) * x0 + (b1 / tot) * x1
    logits = jnp.dot(outp, lw_ref[...], preferred_element_type=jnp.float32)
    logits = jnp.maximum(logits + lb_ref[...], 0.0)
    zm = jnp.max(logits, axis=1, keepdims=True)
    z = logits - zm
    out_ref[...] = z - jnp.log(jnp.sum(jnp.exp(z), axis=1, keepdims=True))


def _final_tc(num2, den2, att_w, lin_w, lin_b):
    hh = num2.shape[2]
    br = 2000
    grid = N // br
    return pl.pallas_call(
        _final_body,
        grid=(grid,),
        in_specs=[
            pl.BlockSpec((2, br, hh), lambda i: (0, i, 0)),
            pl.BlockSpec((2, br, 1), lambda i: (0, i, 0)),
            pl.BlockSpec((hh, 1), lambda i: (0, 0)),
            pl.BlockSpec((hh, NCLASS), lambda i: (0, 0)),
            pl.BlockSpec((1, NCLASS), lambda i: (0, 0)),
        ],
        out_specs=pl.BlockSpec((br, NCLASS), lambda i: (i, 0)),
        out_shape=jax.ShapeDtypeStruct((N, NCLASS), jnp.float32),
    )(num2, den2, att_w, lin_w, lin_b)


def kernel(input, index, node_emb, edge_index_APA, edge_emb_APA,
           edge_index_APCPA, edge_emb_APCPA, n_sample, W1_0, a1_0, W1_1, a1_1,
           W2_0, a2_0, W2_1, a2_1, att_w, lin_W, lin_b):
    x_pad = jnp.concatenate(
        [input, jnp.zeros((NP - N, NFEAT), jnp.float32)], axis=0)
    pad_idx = (jnp.arange(_EP - E, dtype=jnp.int32) % N)

    src = jnp.stack(
        [jnp.concatenate([edge_index_APA[0], pad_idx]),
         jnp.concatenate([edge_index_APCPA[0], pad_idx]) + NP])
    dst = jnp.stack([jnp.concatenate([edge_index_APA[1], pad_idx]),
                     jnp.concatenate([edge_index_APCPA[1], pad_idx])])

    W1 = jnp.stack([W1_0, W1_1])
    W2 = jnp.stack([W2_0, W2_1])
    a1d = jnp.stack([a1_0[:NHID], a1_1[:NHID]])
    a1s = jnp.stack([a1_0[NHID:2 * NHID], a1_1[NHID:2 * NHID]])
    a2d = jnp.stack([a2_0[:DIM_MP], a2_1[:DIM_MP]])
    a2s = jnp.stack([a2_0[DIM_MP:2 * DIM_MP], a2_1[DIM_MP:2 * DIM_MP]])

    e1_0, e2_0, m1_0, m2_0 = _edge_tc(edge_emb_APA,
                                      a1_0[2 * NHID:], a2_0[2 * DIM_MP:])
    e1_1, e2_1, m1_1, m2_1 = _edge_tc(edge_emb_APCPA,
                                      a1_1[2 * NHID:], a2_1[2 * DIM_MP:])
    tail = jnp.full((_EP - E,), -1e30, jnp.float32)
    eed1 = jnp.stack([jnp.concatenate([e1_0.reshape(E), tail]),
                      jnp.concatenate([e1_1.reshape(E), tail])])
    eed2 = jnp.stack([jnp.concatenate([e2_0.reshape(E), tail]),
                      jnp.concatenate([e2_1.reshape(E), tail])])
    me1 = jnp.stack([m1_0[0, 0], m1_1[0, 0]])
    me2 = jnp.stack([m2_0[0, 0], m2_1[0, 0]])

    h1, hd, hs, mhd, mhs = _prep_tc(x_pad, W1, a1d, a1s)
    c1 = jnp.maximum(mhd[:, 0, 0] + mhs[:, 0, 0] + me1, 0.0)
    cv1 = jnp.broadcast_to(c1[:, None], (2, 16))
    num1, den1 = _sc_aggregate(h1, hd.reshape(2, NP), hs.reshape(2, NP),
                               src, dst, eed1, cv1)

    h2, hd2, hs2, mhd2, mhs2 = _next_tc(num1, den1.reshape(2, NP, 1),
                                        W2, a2d, a2s)
    c2 = jnp.maximum(mhd2[:, 0, 0] + mhs2[:, 0, 0] + me2, 0.0)
    cv2 = jnp.broadcast_to(c2[:, None], (2, 16))
    num2, den2 = _sc_aggregate(h2, hd2.reshape(2, NP), hs2.reshape(2, NP),
                               src, dst, eed2, cv2)

    return _final_tc(num2, den2.reshape(2, NP, 1), att_w, lin_W,
                     lin_b.reshape(1, NCLASS))


# speedup vs baseline: 1.4209x; 1.4209x over previous
import functools

import jax
import jax.numpy as jnp
from jax import lax
from jax.experimental import pallas as pl
from jax.experimental.pallas import tpu as pltpu
from jax.experimental.pallas import tpu_sc as plsc

N = 10000
NP = 10240
E = 320000
NFEAT = 128
NHID = 128
DIM_MP = 64
EDGE_DIM = 16
NMETA = 2
NCLASS = 8

_NC = 2
_NS = 16
_EP = 327680
_EPW = _EP // _NS
_CH = 64
_BLKE = 2048
_NCHB = _BLKE // _CH
_NBLK = _EPW // _BLKE
_NRS = NP // _NS


def _prep_body(x_ref, w_ref, ad_ref, as_ref,
               h_ref, hd_ref, hs_ref, mhd_ref, mhs_ref):
    i = pl.program_id(1)
    h = jnp.dot(x_ref[...], w_ref[0], preferred_element_type=jnp.float32)
    h_ref[0] = h
    hd = jnp.dot(h, ad_ref[0], preferred_element_type=jnp.float32)
    hs = jnp.dot(h, as_ref[0], preferred_element_type=jnp.float32)
    hd_ref[0] = hd
    hs_ref[0] = hs
    bmhd = jnp.max(hd).reshape(1, 1, 1)
    bmhs = jnp.max(hs).reshape(1, 1, 1)

    @pl.when(i == 0)
    def _():
        mhd_ref[...] = bmhd
        mhs_ref[...] = bmhs

    @pl.when(i > 0)
    def _():
        mhd_ref[...] = jnp.maximum(mhd_ref[...], bmhd)
        mhs_ref[...] = jnp.maximum(mhs_ref[...], bmhs)


def _prep_tc(x, w, a_d, a_s):
    f = x.shape[1]
    nh = w.shape[2]
    br = 2048
    grid = (2, NP // br)
    return pl.pallas_call(
        _prep_body,
        grid=grid,
        in_specs=[
            pl.BlockSpec((br, f), lambda m, i: (i, 0)),
            pl.BlockSpec((1, f, nh), lambda m, i: (m, 0, 0)),
            pl.BlockSpec((1, nh, 1), lambda m, i: (m, 0, 0)),
            pl.BlockSpec((1, nh, 1), lambda m, i: (m, 0, 0)),
        ],
        out_specs=[
            pl.BlockSpec((1, br, nh), lambda m, i: (m, i, 0)),
            pl.BlockSpec((1, br, 1), lambda m, i: (m, i, 0)),
            pl.BlockSpec((1, br, 1), lambda m, i: (m, i, 0)),
            pl.BlockSpec((1, 1, 1), lambda m, i: (m, 0, 0)),
            pl.BlockSpec((1, 1, 1), lambda m, i: (m, 0, 0)),
        ],
        out_shape=[
            jax.ShapeDtypeStruct((2, NP, nh), jnp.float32),
            jax.ShapeDtypeStruct((2, NP, 1), jnp.float32),
            jax.ShapeDtypeStruct((2, NP, 1), jnp.float32),
            jax.ShapeDtypeStruct((2, 1, 1), jnp.float32),
            jax.ShapeDtypeStruct((2, 1, 1), jnp.float32),
        ],
        name="prep",
    )(x, w, a_d, a_s)


def _next_body(n_ref, d_ref, w_ref, ad_ref, as_ref,
               h_ref, hd_ref, hs_ref, mhd_ref, mhs_ref):
    i = pl.program_id(1)
    x = n_ref[0] / (d_ref[0] + 1e-16)
    x = jnp.where(x > 0, x, jnp.exp(x) - 1.0)
    h = jnp.dot(x, w_ref[0], preferred_element_type=jnp.float32)
    h_ref[0] = h
    hd = jnp.dot(h, ad_ref[0], preferred_element_type=jnp.float32)
    hs = jnp.dot(h, as_ref[0], preferred_element_type=jnp.float32)
    hd_ref[0] = hd
    hs_ref[0] = hs
    bmhd = jnp.max(hd).reshape(1, 1, 1)
    bmhs = jnp.max(hs).reshape(1, 1, 1)

    @pl.when(i == 0)
    def _():
        mhd_ref[...] = bmhd
        mhs_ref[...] = bmhs

    @pl.when(i > 0)
    def _():
        mhd_ref[...] = jnp.maximum(mhd_ref[...], bmhd)
        mhs_ref[...] = jnp.maximum(mhs_ref[...], bmhs)


def _next_tc(num, den, w, a_d, a_s):
    hin = num.shape[2]
    nh = w.shape[2]
    br = 2048
    grid = (2, NP // br)
    return pl.pallas_call(
        _next_body,
        grid=grid,
        in_specs=[
            pl.BlockSpec((1, br, hin), lambda m, i: (m, i, 0)),
            pl.BlockSpec((1, br, 1), lambda m, i: (m, i, 0)),
            pl.BlockSpec((1, hin, nh), lambda m, i: (m, 0, 0)),
            pl.BlockSpec((1, nh, 1), lambda m, i: (m, 0, 0)),
            pl.BlockSpec((1, nh, 1), lambda m, i: (m, 0, 0)),
        ],
        out_specs=[
            pl.BlockSpec((1, br, nh), lambda m, i: (m, i, 0)),
            pl.BlockSpec((1, br, 1), lambda m, i: (m, i, 0)),
            pl.BlockSpec((1, br, 1), lambda m, i: (m, i, 0)),
            pl.BlockSpec((1, 1, 1), lambda m, i: (m, 0, 0)),
            pl.BlockSpec((1, 1, 1), lambda m, i: (m, 0, 0)),
        ],
        out_shape=[
            jax.ShapeDtypeStruct((2, NP, nh), jnp.float32),
            jax.ShapeDtypeStruct((2, NP, 1), jnp.float32),
            jax.ShapeDtypeStruct((2, NP, 1), jnp.float32),
            jax.ShapeDtypeStruct((2, 1, 1), jnp.float32),
            jax.ShapeDtypeStruct((2, 1, 1), jnp.float32),
        ],
        name="nextprep",
    )(num, den, w, a_d, a_s)


def _edge_body(ee_ref, a1_ref, a2_ref, d1_ref, d2_ref, m1_ref, m2_ref):
    i = pl.program_id(0)
    d1 = jnp.dot(ee_ref[...], a1_ref[...], preferred_element_type=jnp.float32)
    d2 = jnp.dot(ee_ref[...], a2_ref[...], preferred_element_type=jnp.float32)
    d1_ref[...] = d1
    d2_ref[...] = d2
    b1 = jnp.max(d1).reshape(1, 1)
    b2 = jnp.max(d2).reshape(1, 1)

    @pl.when(i == 0)
    def _():
        m1_ref[...] = b1
        m2_ref[...] = b2

    @pl.when(i > 0)
    def _():
        m1_ref[...] = jnp.maximum(m1_ref[...], b1)
        m2_ref[...] = jnp.maximum(m2_ref[...], b2)


def _edge_tc(eev, a1bd, a2bd):
    er = E // 8
    br = 5000
    grid = er // br
    return pl.pallas_call(
        _edge_body,
        grid=(grid,),
        in_specs=[
            pl.BlockSpec((br, 128), lambda i: (i, 0)),
            pl.BlockSpec((128, 8), lambda i: (0, 0)),
            pl.BlockSpec((128, 8), lambda i: (0, 0)),
        ],
        out_specs=[
            pl.BlockSpec((br, 8), lambda i: (i, 0)),
            pl.BlockSpec((br, 8), lambda i: (i, 0)),
            pl.BlockSpec((1, 1), lambda i: (0, 0)),
            pl.BlockSpec((1, 1), lambda i: (0, 0)),
        ],
        out_shape=[
            jax.ShapeDtypeStruct((er, 8), jnp.float32),
            jax.ShapeDtypeStruct((er, 8), jnp.float32),
            jax.ShapeDtypeStruct((1, 1), jnp.float32),
            jax.ShapeDtypeStruct((1, 1), jnp.float32),
        ],
        name="edge_dot",
    )(eev, a1bd, a2bd)


def _blockdiag8(a_e):
    z = jnp.zeros((8, 16, 8), jnp.float32)
    z = z.at[jnp.arange(8), :, jnp.arange(8)].set(a_e[:, 0][None, :])
    return z.reshape(128, 8)


def _sc_body(hh,
             h_ref, hd_ref, hs_ref, src_ref, dst_ref, eed_ref,
             cv_ref, num_ref, den_ref,
             hd_v, hs_v, cv, srcb, dstb, eedb, dstc, wc, rows0, rows1,
             zden, num_sp, den_sp, gsem0, gsem1, ssem, dsem):
    ci = lax.axis_index("c")
    s = lax.axis_index("s")

    pltpu.sync_copy(hd_ref.at[ci], hd_v)
    pltpu.sync_copy(hs_ref.at[ci], hs_v)
    pltpu.sync_copy(cv_ref.at[ci], cv)

    nfv = hh // 16
    ngr = _CH // 16

    @plsc.parallel_loop(0, _CH)
    def _(r):
        for f in range(nfv):
            rows0[r, pl.ds(f * 16, 16)] = jnp.zeros((16,), jnp.float32)

    @plsc.parallel_loop(0, _NRS, step=16)
    def _(r):
        zden[pl.ds(r, 16)] = jnp.zeros((16,), jnp.float32)

    for k in range(_NRS // _CH):
        pltpu.sync_copy(rows0, num_sp.at[pl.ds(s * _NRS + k * _CH, _CH)])
    pltpu.sync_copy(zden, den_sp.at[pl.ds(s * _NRS, _NRS)])

    plsc.subcore_barrier()

    cvv = cv[...]
    wbase = s * _EPW
    coff = ci * NP

    def score(c):
        off = c * _CH
        for g in range(ngr):
            o = off + g * 16
            dv = dstb[pl.ds(o, 16)]
            sv = srcb[pl.ds(o, 16)] - coff
            ev = eedb[pl.ds(o, 16)]
            hdv = plsc.load_gather(hd_v, [dv])
            hsv = plsc.load_gather(hs_v, [sv])
            t = hdv + hsv + ev
            e = jnp.maximum(t, t * 0.2)
            w = jnp.exp(e - cvv)
            wc[c, pl.ds(g * 16, 16)] = w
            dstc[c, pl.ds(g * 16, 16)] = dv

    def gissue(c, rbuf, gsem):
        pltpu.async_copy(h_ref.at[srcb.at[pl.ds(c * _CH, _CH)]], rbuf, gsem)

    def scale(c, rbuf):
        @plsc.parallel_loop(0, ngr)
        def _(g):
            wv16 = wc[c, pl.ds(g * 16, 16)]
            for j in range(16):
                wb = jnp.full((16,), wv16[j], jnp.float32)
                r = g * 16 + j
                for f in range(nfv):
                    sl = pl.ds(f * 16, 16)
                    rbuf[r, sl] = rbuf[r, sl] * wb

    def swait(rbuf):
        pltpu.make_async_copy(rbuf, num_sp.at[dstc.at[0]], ssem).wait()

    def dwait():
        pltpu.make_async_copy(wc.at[0], den_sp.at[dstc.at[0]], dsem).wait()

    def block_body(b, carry):
        @pl.when(b > 0)
        def _():
            swait(rows0)
            dwait()

        base = wbase + b * _BLKE
        pltpu.sync_copy(src_ref.at[ci, pl.ds(base, _BLKE)], srcb)
        pltpu.sync_copy(dst_ref.at[ci, pl.ds(base, _BLKE)], dstb)
        pltpu.sync_copy(eed_ref.at[ci, pl.ds(base, _BLKE)], eedb)

        score(0)
        gissue(0, rows0, gsem0)

        def chunk_body(c, carry):
            even = (c & 1) == 0

            @pl.when(c >= 1)
            def _():
                swait(rows0)
                dwait()

            @pl.when(c < _NCHB - 1)
            def _():
                score(c + 1)

                @pl.when(even)
                def _():
                    gissue(c + 1, rows1, gsem1)

                @pl.when(jnp.logical_not(even))
                def _():
                    gissue(c + 1, rows0, gsem0)

            @pl.when(even)
            def _():
                pltpu.make_async_copy(
                    h_ref.at[srcb.at[pl.ds(0, _CH)]], rows0, gsem0).wait()
                scale(c, rows0)
                pltpu.async_copy(rows0, num_sp.at[dstc.at[c]], ssem, add=True)

            @pl.when(jnp.logical_not(even))
            def _():
                pltpu.make_async_copy(
                    h_ref.at[srcb.at[pl.ds(0, _CH)]], rows1, gsem1).wait()
                scale(c, rows1)
                pltpu.async_copy(rows1, num_sp.at[dstc.at[c]], ssem, add=True)

            pltpu.async_copy(wc.at[c], den_sp.at[dstc.at[c]], dsem, add=True)
            return carry

        lax.fori_loop(0, _NCHB, chunk_body, 0)
        return carry

    lax.fori_loop(0, _NBLK, block_body, 0)

    swait(rows0)
    dwait()

    plsc.subcore_barrier()

    r0 = s * _NRS
    pltpu.sync_copy(num_sp.at[pl.ds(r0, _NRS)], num_ref.at[ci, pl.ds(r0, _NRS)])
    pltpu.sync_copy(den_sp.at[pl.ds(r0, _NRS)], den_ref.at[ci, pl.ds(r0, _NRS)])


def _sc_aggregate(h, hd, hs, src, dst, eed, cvec):
    hh = h.shape[2]
    mesh = plsc.VectorSubcoreMesh(core_axis_name="c", subcore_axis_name="s")
    f = pl.kernel(
        functools.partial(_sc_body, hh),
        out_type=[
            jax.ShapeDtypeStruct((2, NP, hh), jnp.float32),
            jax.ShapeDtypeStruct((2, NP), jnp.float32),
        ],
        mesh=mesh,
        scratch_types=[
            pltpu.VMEM((NP,), jnp.float32),
            pltpu.VMEM((NP,), jnp.float32),
            pltpu.VMEM((16,), jnp.float32),
            pltpu.VMEM((_BLKE,), jnp.int32),
            pltpu.VMEM((_BLKE,), jnp.int32),
            pltpu.VMEM((_BLKE,), jnp.float32),
            pltpu.VMEM((_NCHB, _CH), jnp.int32),
            pltpu.VMEM((_NCHB, _CH), jnp.float32),
            pltpu.VMEM((_CH, hh), jnp.float32),
            pltpu.VMEM((_CH, hh), jnp.float32),
            pltpu.VMEM((_NRS,), jnp.float32),
            pltpu.VMEM_SHARED((NP, hh), jnp.float32),
            pltpu.VMEM_SHARED((NP,), jnp.float32),
            pltpu.SemaphoreType.DMA,
            pltpu.SemaphoreType.DMA,
            pltpu.SemaphoreType.DMA,
            pltpu.SemaphoreType.DMA,
        ],
        compiler_params=pltpu.CompilerParams(
            needs_layout_passes=False, use_tc_tiling_on_sc=False),
        name="sc_agg",
    )
    return f(h.reshape(2 * NP, hh), hd, hs, src, dst, eed, cvec)


def _final_body(n_ref, d_ref, aw_ref, lw_ref, lb_ref, out_ref):
    x0 = n_ref[0] / (d_ref[0] + 1e-16)
    x0 = jnp.where(x0 > 0, x0, jnp.exp(x0) - 1.0)
    x1 = n_ref[1] / (d_ref[1] + 1e-16)
    x1 = jnp.where(x1 > 0, x1, jnp.exp(x1) - 1.0)
    s0 = jnp.tanh(jnp.dot(x0, aw_ref[...], preferred_element_type=jnp.float32))
    s1 = jnp.tanh(jnp.dot(x1, aw_ref[...], preferred_element_type=jnp.float32))
    m = jnp.maximum(s0, s1)
    b0 = jnp.exp(s0 - m)
    b1 = jnp.exp(s1 - m)
    tot = b0 + b1
    outp = (b0 / tot) * x0 + (b1 / tot) * x1
    logits = jnp.dot(outp, lw_ref[...], preferred_element_type=jnp.float32)
    logits = jnp.maximum(logits + lb_ref[...], 0.0)
    zm = jnp.max(logits, axis=1, keepdims=True)
    z = logits - zm
    out_ref[...] = z - jnp.log(jnp.sum(jnp.exp(z), axis=1, keepdims=True))


def _final_tc(num2, den2, att_w, lin_w, lin_b):
    hh = num2.shape[2]
    br = 2000
    grid = N // br
    return pl.pallas_call(
        _final_body,
        grid=(grid,),
        in_specs=[
            pl.BlockSpec((2, br, hh), lambda i: (0, i, 0)),
            pl.BlockSpec((2, br, 1), lambda i: (0, i, 0)),
            pl.BlockSpec((hh, 1), lambda i: (0, 0)),
            pl.BlockSpec((hh, NCLASS), lambda i: (0, 0)),
            pl.BlockSpec((1, NCLASS), lambda i: (0, 0)),
        ],
        out_specs=pl.BlockSpec((br, NCLASS), lambda i: (i, 0)),
        out_shape=jax.ShapeDtypeStruct((N, NCLASS), jnp.float32),
        name="final",
    )(num2, den2, att_w, lin_w, lin_b)


def kernel(input, index, node_emb, edge_index_APA, edge_emb_APA,
           edge_index_APCPA, edge_emb_APCPA, n_sample, W1_0, a1_0, W1_1, a1_1,
           W2_0, a2_0, W2_1, a2_1, att_w, lin_W, lin_b):
    x_pad = jnp.concatenate(
        [input, jnp.zeros((NP - N, NFEAT), jnp.float32)], axis=0)
    pad_idx = (jnp.arange(_EP - E, dtype=jnp.int32) % N)

    src = jnp.stack(
        [jnp.concatenate([edge_index_APA[0], pad_idx]),
         jnp.concatenate([edge_index_APCPA[0], pad_idx]) + NP])
    dst = jnp.stack([jnp.concatenate([edge_index_APA[1], pad_idx]),
                     jnp.concatenate([edge_index_APCPA[1], pad_idx])])

    W1 = jnp.stack([W1_0, W1_1])
    W2 = jnp.stack([W2_0, W2_1])
    a1d = jnp.stack([a1_0[:NHID], a1_1[:NHID]])
    a1s = jnp.stack([a1_0[NHID:2 * NHID], a1_1[NHID:2 * NHID]])
    a2d = jnp.stack([a2_0[:DIM_MP], a2_1[:DIM_MP]])
    a2s = jnp.stack([a2_0[DIM_MP:2 * DIM_MP], a2_1[DIM_MP:2 * DIM_MP]])

    e1_0, e2_0, m1_0, m2_0 = _edge_tc(
        edge_emb_APA.reshape(E // 8, 128),
        _blockdiag8(a1_0[2 * NHID:]), _blockdiag8(a2_0[2 * DIM_MP:]))
    e1_1, e2_1, m1_1, m2_1 = _edge_tc(
        edge_emb_APCPA.reshape(E // 8, 128),
        _blockdiag8(a1_1[2 * NHID:]), _blockdiag8(a2_1[2 * DIM_MP:]))
    tail = jnp.full((_EP - E,), -1e30, jnp.float32)
    eed1 = jnp.stack([jnp.concatenate([e1_0.reshape(E), tail]),
                      jnp.concatenate([e1_1.reshape(E), tail])])
    eed2 = jnp.stack([jnp.concatenate([e2_0.reshape(E), tail]),
                      jnp.concatenate([e2_1.reshape(E), tail])])
    me1 = jnp.stack([m1_0[0, 0], m1_1[0, 0]])
    me2 = jnp.stack([m2_0[0, 0], m2_1[0, 0]])

    h1, hd, hs, mhd, mhs = _prep_tc(x_pad, W1, a1d, a1s)
    c1 = jnp.maximum(mhd[:, 0, 0] + mhs[:, 0, 0] + me1, 0.0)
    cv1 = jnp.broadcast_to(c1[:, None], (2, 16))
    num1, den1 = _sc_aggregate(h1, hd.reshape(2, NP), hs.reshape(2, NP),
                               src, dst, eed1, cv1)

    h2, hd2, hs2, mhd2, mhs2 = _next_tc(num1, den1.reshape(2, NP, 1),
                                        W2, a2d, a2s)
    c2 = jnp.maximum(mhd2[:, 0, 0] + mhs2[:, 0, 0] + me2, 0.0)
    cv2 = jnp.broadcast_to(c2[:, None], (2, 16))
    num2, den2 = _sc_aggregate(h2, hd2.reshape(2, NP), hs2.reshape(2, NP),
                               src, dst, eed2, cv2)

    return _final_tc(num2, den2.reshape(2, NP, 1), att_w, lin_W,
                     lin_b.reshape(1, NCLASS))
